# Initial kernel scaffold; baseline (speedup 1.0000x reference)
import functools

import jax
import jax.numpy as jnp
from jax import lax
from jax.experimental import pallas as pl
from jax.experimental.pallas import tpu as pltpu
from jax.experimental.pallas import tpu_sc as plsc

_IDXW = 128


@functools.lru_cache(maxsize=None)
def _build_gather(R, V, D, NC, NS):
    NW = NC * NS
    rows_per_w = R // NW
    G = 8
    n_chunks = rows_per_w // G

    mesh = plsc.VectorSubcoreMesh(core_axis_name="c", subcore_axis_name="s")

    @functools.partial(
        pl.kernel,
        out_type=jax.ShapeDtypeStruct((R, _IDXW, D), jnp.float32),
        mesh=mesh,
        scratch_types=[
            pltpu.VMEM((G, _IDXW), jnp.int32),
            pltpu.VMEM((G, _IDXW, D), jnp.float32),
            pltpu.SemaphoreType.DMA,
        ],
    )
    def gather_kernel(idx_hbm, table_hbm, out_hbm, idx_v, rows_v, sem):
        wid = lax.axis_index("s") * NC + lax.axis_index("c")
        base = wid * rows_per_w

        @pl.loop(0, n_chunks)
        def _chunk(i):
            off = base + i * G
            pltpu.sync_copy(idx_hbm.at[pl.ds(off, G)], idx_v)
            copies = [
                pltpu.async_copy(table_hbm.at[idx_v.at[j]], rows_v.at[j], sem)
                for j in range(G)
            ]
            for c in copies:
                c.wait()
            pltpu.sync_copy(rows_v, out_hbm.at[pl.ds(off, G)])

    return gather_kernel


def kernel(tokens, table):
    B, H = tokens.shape
    V, D = table.shape
    N = B * H
    idx2d = tokens.reshape(N // _IDXW, _IDXW).astype(jnp.int32)
    info = plsc.get_sparse_core_info()
    fn = _build_gather(N // _IDXW, V, D, info.num_cores, info.num_subcores)
    out = fn(idx2d, table)
    return out.reshape(B, H, D)


# speedup vs baseline: 1.4580x; 1.4580x over previous
import functools

import jax
import jax.numpy as jnp
from jax import lax
from jax.experimental import pallas as pl
from jax.experimental.pallas import tpu as pltpu
from jax.experimental.pallas import tpu_sc as plsc

_IDXW = 128


@functools.lru_cache(maxsize=None)
def _build_gather(R, V, D, NC, NS):
    NW = NC * NS
    rows_per_w = R // NW
    G = 8
    n_chunks = rows_per_w // G

    mesh = plsc.VectorSubcoreMesh(core_axis_name="c", subcore_axis_name="s")

    @functools.partial(
        pl.kernel,
        out_type=jax.ShapeDtypeStruct((R, _IDXW, D), jnp.float32),
        mesh=mesh,
        scratch_types=[
            pltpu.VMEM((G, _IDXW), jnp.int32),
            pltpu.VMEM((G, _IDXW, D), jnp.float32),
            pltpu.SemaphoreType.DMA,
        ],
        compiler_params=pltpu.CompilerParams(use_tc_tiling_on_sc=False),
    )
    def gather_kernel(idx_hbm, table_hbm, out_hbm, idx_v, rows_v, sem):
        wid = lax.axis_index("s") * NC + lax.axis_index("c")
        base = wid * rows_per_w

        @pl.loop(0, n_chunks)
        def _chunk(i):
            off = base + i * G
            pltpu.sync_copy(idx_hbm.at[pl.ds(off, G)], idx_v)
            copies = [
                pltpu.async_copy(table_hbm.at[idx_v.at[j]], rows_v.at[j], sem)
                for j in range(G)
            ]
            for c in copies:
                c.wait()
            pltpu.sync_copy(rows_v, out_hbm.at[pl.ds(off, G)])

    return gather_kernel


def kernel(tokens, table):
    B, H = tokens.shape
    V, D = table.shape
    N = B * H
    idx2d = tokens.reshape(N // _IDXW, _IDXW).astype(jnp.int32)
    info = plsc.get_sparse_core_info()
    fn = _build_gather(N // _IDXW, V, D, info.num_cores, info.num_subcores)
    out = fn(idx2d, table)
    return out.reshape(B, H, D)


# speedup vs baseline: 1.4949x; 1.0253x over previous
import functools

import jax
import jax.numpy as jnp
from jax import lax
from jax.experimental import pallas as pl
from jax.experimental.pallas import tpu as pltpu
from jax.experimental.pallas import tpu_sc as plsc

_IDXW = 128


@functools.lru_cache(maxsize=None)
def _build_gather(R, V, D, NC, NS):
    NW = NC * NS
    rows_per_w = R // NW
    G = 10
    NBUF = 2
    n_chunks = rows_per_w // G
    n_outer = n_chunks // NBUF

    mesh = plsc.VectorSubcoreMesh(core_axis_name="c", subcore_axis_name="s")

    @functools.partial(
        pl.kernel,
        out_type=jax.ShapeDtypeStruct((R, _IDXW, D), jnp.float32),
        mesh=mesh,
        scratch_types=[
            pltpu.VMEM((NBUF, G, _IDXW), jnp.int32),
            pltpu.VMEM((NBUF, G, _IDXW, D), jnp.float32),
            pltpu.SemaphoreType.DMA,
            pltpu.SemaphoreType.DMA,
            pltpu.SemaphoreType.DMA,
        ],
        compiler_params=pltpu.CompilerParams(use_tc_tiling_on_sc=False),
    )
    def gather_kernel(idx_hbm, table_hbm, out_hbm, idx_v, rows_v, sem_idx,
                      sem_gat, sem_st):
        wid = lax.axis_index("s") * NC + lax.axis_index("c")
        base = wid * rows_per_w

        for b in range(NBUF):
            pltpu.async_copy(idx_hbm.at[pl.ds(base + b * G, G)],
                             idx_v.at[b], sem_idx)

        @pl.loop(0, n_outer)
        def _outer(k):
            for b in range(NBUF):
                off = base + (k * NBUF + b) * G

                pltpu.make_async_copy(idx_hbm.at[pl.ds(base, G)],
                                      idx_v.at[b], sem_idx).wait()

                @pl.when(k >= 1)
                def _():
                    pltpu.make_async_copy(rows_v.at[b],
                                          out_hbm.at[pl.ds(base, G)],
                                          sem_st).wait()

                for j in range(G):
                    pltpu.async_copy(table_hbm.at[idx_v.at[b].at[j]],
                                     rows_v.at[b].at[j], sem_gat)

                pltpu.make_async_copy(out_hbm.at[pl.ds(base, G)],
                                      rows_v.at[b], sem_gat).wait()

                @pl.when(k < n_outer - 1)
                def _():
                    pltpu.async_copy(idx_hbm.at[pl.ds(off + NBUF * G, G)],
                                     idx_v.at[b], sem_idx)

                pltpu.async_copy(rows_v.at[b], out_hbm.at[pl.ds(off, G)],
                                 sem_st)

        for b in range(NBUF):
            pltpu.make_async_copy(rows_v.at[b], out_hbm.at[pl.ds(base, G)],
                                  sem_st).wait()

    return gather_kernel


def kernel(tokens, table):
    B, H = tokens.shape
    V, D = table.shape
    N = B * H
    idx2d = tokens.reshape(N // _IDXW, _IDXW).astype(jnp.int32)
    info = plsc.get_sparse_core_info()
    fn = _build_gather(N // _IDXW, V, D, info.num_cores, info.num_subcores)
    out = fn(idx2d, table)
    return out.reshape(B, H, D)
